# baseline (speedup 1.0000x reference)
import functools

import jax
import jax.numpy as jnp
from jax import lax
from jax.experimental import pallas as pl
from jax.experimental.pallas import tpu as pltpu
from jax.experimental.pallas import tpu_sc as plsc

_NC, _NS = 2, 16
_NW = _NC * _NS
_B = 1024 * 20
_PER_W = _B // _NW
_CHUNK = 32
_NCHUNK = _PER_W // _CHUNK
_NPAIR = _NCHUNK // 2

_D_BIG = 768
_D_SMALL = 256
_D_OUT = 2 * _D_BIG + 3 * _D_SMALL


def _build():
  mesh = plsc.VectorSubcoreMesh(core_axis_name="c", subcore_axis_name="s")

  @functools.partial(
      pl.kernel, mesh=mesh,
      out_type=jax.ShapeDtypeStruct((_B, _D_OUT), jnp.float32),
      scratch_types=[
          pltpu.VMEM((_NCHUNK, _CHUNK), jnp.int32),
          pltpu.VMEM((_CHUNK, _D_BIG), jnp.float32),
          pltpu.VMEM((_CHUNK, _D_BIG), jnp.float32),
          pltpu.VMEM((_CHUNK, _D_SMALL), jnp.float32),
          pltpu.VMEM((_CHUNK, _D_SMALL), jnp.float32),
          pltpu.SemaphoreType.DMA,
          pltpu.SemaphoreType.DMA,
          pltpu.SemaphoreType.DMA,
          pltpu.SemaphoreType.DMA,
      ],
  )
  def emb(ip, ic, icol, isz, igr, wp, wc, wcol, wsz, wgr,
          out, idx_v, big0, big1, small0, small1, gs0, gs1, ws0, ws1):
    wid = lax.axis_index("s") * _NC + lax.axis_index("c")
    base = wid * _PER_W
    tables = [
        (ip, wp, big0, big1, _D_BIG, 0),
        (ic, wc, big0, big1, _D_BIG, _D_BIG),
        (icol, wcol, small0, small1, _D_SMALL, 2 * _D_BIG),
        (isz, wsz, small0, small1, _D_SMALL, 2 * _D_BIG + _D_SMALL),
        (igr, wgr, small0, small1, _D_SMALL, 2 * _D_BIG + 2 * _D_SMALL),
    ]
    for idx_hbm, w_hbm, buf0, buf1, dcol, coff in tables:
      pltpu.sync_copy(idx_hbm.at[wid], idx_v)

      def gather(c, buf, sem, w_hbm=w_hbm):
        pltpu.async_copy(w_hbm.at[idx_v.at[c]], buf, sem)

      def gwait(buf, sem, w_hbm=w_hbm):
        pltpu.make_async_copy(w_hbm.at[idx_v.at[0]], buf, sem).wait()

      def odst(c, dcol=dcol, coff=coff):
        return out.at[pl.ds(base + c * _CHUNK, _CHUNK), pl.ds(coff, dcol)]

      def write(c, buf, sem):
        pltpu.async_copy(buf, odst(c), sem)

      def wwait(buf, sem):
        pltpu.make_async_copy(buf, odst(0), sem).wait()

      gather(0, buf0, gs0)

      def body(k, carry, buf0=buf0, buf1=buf1):
        c0 = 2 * k
        gwait(buf0, gs0)
        gather(c0 + 1, buf1, gs1)
        write(c0, buf0, ws0)
        gwait(buf1, gs1)
        wwait(buf0, ws0)
        gather(c0 + 2, buf0, gs0)
        write(c0 + 1, buf1, ws1)
        wwait(buf1, ws1)
        return carry

      lax.fori_loop(0, _NPAIR - 1, body, None)

      c0 = _NCHUNK - 2
      gwait(buf0, gs0)
      gather(c0 + 1, buf1, gs1)
      write(c0, buf0, ws0)
      gwait(buf1, gs1)
      wwait(buf0, ws0)
      write(c0 + 1, buf1, ws1)
      wwait(buf1, ws1)

  return emb


_EMB = _build()


def kernel(product, customer, color, size, group,
           W_product, W_customer, W_color, W_size, W_group):
  def prep(i):
    return jnp.asarray(i, jnp.int32).reshape(_NW, _NCHUNK, _CHUNK)

  out = _EMB(prep(product), prep(customer), prep(color), prep(size),
             prep(group), W_product, W_customer, W_color, W_size, W_group)
  return out.reshape(1024, 20, _D_OUT)
